# baseline (speedup 1.0000x reference)
import functools

import jax
import jax.numpy as jnp
from jax import lax
from jax.experimental import pallas as pl
from jax.experimental.pallas import tpu as pltpu
from jax.experimental.pallas import tpu_sc as plsc

NUM_CLASSES = 2048
DIM = 128
FEAT = 3 * DIM
BATCH = 1024
N_RES = 256
MAX_LABELS = 20

NC = 2
NS = 16
L = 16
NW = NC * NS
B_PER_W = BATCH // NW
IDX_PER_W = B_PER_W * MAX_LABELS
IDX_CHUNK = 128
N_CHUNKS = IDX_PER_W // IDX_CHUNK

@functools.lru_cache(maxsize=1)
def _make_pool_sc():
    mesh = plsc.VectorSubcoreMesh(
        core_axis_name="c", subcore_axis_name="s", num_cores=NC, num_subcores=NS
    )

    @functools.partial(
        pl.kernel,
        out_type=jax.ShapeDtypeStruct((BATCH, FEAT), jnp.float32),
        mesh=mesh,
        scratch_types=[
            pltpu.VMEM((8, IDX_CHUNK), jnp.int32),
            pltpu.VMEM((IDX_PER_W, DIM), jnp.float32),
            pltpu.VMEM((B_PER_W, FEAT), jnp.float32),
            pltpu.SemaphoreType.DMA,
        ],
    )
    def _pool_sc(idx_c_hbm, idx_a_hbm, idx_t_hbm, emb_c_hbm, emb_a_hbm,
                 emb_t_hbm, out_hbm, idx_v, rows_v, out_v, sem):
        wid = lax.axis_index("s") * NC + lax.axis_index("c")
        inv = jnp.float32(1.0 / MAX_LABELS)

        for t, (idx_hbm, emb_hbm) in enumerate(
            ((idx_c_hbm, emb_c_hbm), (idx_a_hbm, emb_a_hbm), (idx_t_hbm, emb_t_hbm))
        ):
            pltpu.sync_copy(idx_hbm.at[wid], idx_v)
            copies = [
                pltpu.async_copy(
                    emb_hbm.at[idx_v.at[j]],
                    rows_v.at[pl.ds(j * IDX_CHUNK, IDX_CHUNK)],
                    sem,
                )
                for j in range(N_CHUNKS)
            ]
            for cp in copies:
                cp.wait()

            def body(b, _, t=t):
                for g in range(DIM // L):
                    col = pl.ds(g * L, L)
                    acc = rows_v[b * MAX_LABELS, col]
                    for lbl in range(1, MAX_LABELS):
                        acc = acc + rows_v[b * MAX_LABELS + lbl, col]
                    out_v[b, pl.ds(t * DIM + g * L, L)] = acc * inv
                return 0

            lax.fori_loop(0, B_PER_W, body, 0)

        pltpu.sync_copy(out_v, out_hbm.at[pl.ds(wid * B_PER_W, B_PER_W)])

    return _pool_sc


_BB = 8


def _bcast_body(pooled_ref, out_ref):
    out_ref[...] = jnp.broadcast_to(
        pooled_ref[...][:, None, :], (_BB, N_RES, FEAT)
    )


_bcast = pl.pallas_call(
    _bcast_body,
    grid=(BATCH // _BB,),
    in_specs=[pl.BlockSpec((_BB, FEAT), lambda i: (i, 0))],
    out_specs=pl.BlockSpec((_BB, N_RES, FEAT), lambda i: (i, 0, 0)),
    out_shape=jax.ShapeDtypeStruct((BATCH, N_RES, FEAT), jnp.float32),
)


def kernel(x_t, cath_code_indices, cath_code_indices_mask, emb_C, emb_A, emb_T):
    idx = cath_code_indices.astype(jnp.int32)

    def _slab(a):
        a = a.reshape(NW, N_CHUNKS, IDX_CHUNK)
        return jnp.pad(a, ((0, 0), (0, 8 - N_CHUNKS), (0, 0)))

    idx_c = _slab(idx[:, :, 0])
    idx_a = _slab(idx[:, :, 1])
    idx_t = _slab(idx[:, :, 2])
    pooled = _make_pool_sc()(idx_c, idx_a, idx_t, emb_C, emb_A, emb_T)
    return _bcast(pooled)


# speedup vs baseline: 1.0973x; 1.0973x over previous
import functools

import jax
import jax.numpy as jnp
from jax import lax
from jax.experimental import pallas as pl
from jax.experimental.pallas import tpu as pltpu
from jax.experimental.pallas import tpu_sc as plsc

NUM_CLASSES = 2048
DIM = 128
FEAT = 3 * DIM
BATCH = 1024
HALF = BATCH // 2
N_RES = 256
MAX_LABELS = 20

NC = 2
NS = 16
L = 16
NW = NC * NS
N_CHUNKS = 5


@functools.lru_cache(maxsize=2)
def _make_pool_sc(n_batch):
    b_per_w = n_batch // NW
    idx_per_w = b_per_w * MAX_LABELS
    idx_chunk = idx_per_w // N_CHUNKS
    b_per_chunk = b_per_w // N_CHUNKS

    mesh = plsc.VectorSubcoreMesh(
        core_axis_name="c", subcore_axis_name="s", num_cores=NC, num_subcores=NS
    )

    @functools.partial(
        pl.kernel,
        out_type=jax.ShapeDtypeStruct((n_batch, FEAT), jnp.float32),
        mesh=mesh,
        scratch_types=[
            pltpu.VMEM((8, idx_chunk), jnp.int32),
            pltpu.VMEM((idx_per_w, DIM), jnp.float32),
            pltpu.VMEM((b_per_w, FEAT), jnp.float32),
            pltpu.SemaphoreType.DMA,
        ],
    )
    def _pool_sc(idx_c_hbm, idx_a_hbm, idx_t_hbm, emb_c_hbm, emb_a_hbm,
                 emb_t_hbm, out_hbm, idx_v, rows_v, out_v, sem):
        wid = lax.axis_index("s") * NC + lax.axis_index("c")
        inv = jnp.float32(1.0 / MAX_LABELS)

        for t, (idx_hbm, emb_hbm) in enumerate(
            ((idx_c_hbm, emb_c_hbm), (idx_a_hbm, emb_a_hbm), (idx_t_hbm, emb_t_hbm))
        ):
            pltpu.sync_copy(idx_hbm.at[wid], idx_v)
            copies = [
                pltpu.async_copy(
                    emb_hbm.at[idx_v.at[j]],
                    rows_v.at[pl.ds(j * idx_chunk, idx_chunk)],
                    sem,
                )
                for j in range(N_CHUNKS)
            ]
            for cp in copies:
                cp.wait()

            def body(b, _, t=t):
                for g in range(DIM // L):
                    col = pl.ds(g * L, L)
                    acc = rows_v[b * MAX_LABELS, col]
                    for lbl in range(1, MAX_LABELS):
                        acc = acc + rows_v[b * MAX_LABELS + lbl, col]
                    out_v[b, pl.ds(t * DIM + g * L, L)] = acc * inv
                return 0

            lax.fori_loop(0, b_per_w, body, 0)

        pltpu.sync_copy(out_v, out_hbm.at[pl.ds(wid * b_per_w, b_per_w)])

    return _pool_sc


_BB = 16
_OUT_SHAPE = jax.ShapeDtypeStruct((BATCH, N_RES, FEAT), jnp.float32)


def _bcast_lo_body(pooled_ref, out_ref):
    out_ref[...] = jnp.broadcast_to(
        pooled_ref[...][:, None, :], (_BB, N_RES, FEAT)
    )


def _bcast_hi_body(buf_ref, pooled_ref, out_ref):
    del buf_ref
    out_ref[...] = jnp.broadcast_to(
        pooled_ref[...][:, None, :], (_BB, N_RES, FEAT)
    )


_bcast_lo = pl.pallas_call(
    _bcast_lo_body,
    grid=(HALF // _BB,),
    in_specs=[pl.BlockSpec((_BB, FEAT), lambda i: (i, 0))],
    out_specs=pl.BlockSpec((_BB, N_RES, FEAT), lambda i: (i, 0, 0)),
    out_shape=_OUT_SHAPE,
)

_bcast_hi = pl.pallas_call(
    _bcast_hi_body,
    grid=(HALF // _BB,),
    in_specs=[
        pl.BlockSpec(memory_space=pl.ANY),
        pl.BlockSpec((_BB, FEAT), lambda i: (i, 0)),
    ],
    out_specs=pl.BlockSpec((_BB, N_RES, FEAT), lambda i: (i + HALF // _BB, 0, 0)),
    out_shape=_OUT_SHAPE,
    input_output_aliases={0: 0},
)


def _slab(a, n_batch):
    idx_chunk = n_batch * MAX_LABELS // (NW * N_CHUNKS)
    a = a.reshape(NW, N_CHUNKS, idx_chunk)
    return jnp.pad(a, ((0, 0), (0, 8 - N_CHUNKS), (0, 0)))


def kernel(x_t, cath_code_indices, cath_code_indices_mask, emb_C, emb_A, emb_T):
    idx = cath_code_indices.astype(jnp.int32)
    pool = _make_pool_sc(HALF)
    pooled = []
    for h in range(2):
        part = idx[h * HALF:(h + 1) * HALF]
        pooled.append(
            pool(
                _slab(part[:, :, 0], HALF),
                _slab(part[:, :, 1], HALF),
                _slab(part[:, :, 2], HALF),
                emb_C, emb_A, emb_T,
            )
        )
    out = _bcast_lo(pooled[0])
    return _bcast_hi(out, pooled[1])
